# baseline (speedup 1.0000x reference)
import functools

import jax
import jax.numpy as jnp
from jax import lax
from jax.experimental import pallas as pl
from jax.experimental.pallas import tpu as pltpu
from jax.experimental.pallas import tpu_sc as plsc

BATCH = 4096
HIST = 200
DIM = 64
N = BATCH * HIST

NC = 2
NS = 16
NW = NC * NS
PER_W = N // NW

G = 128
GPC = 5
CHUNK = G * GPC
NCHUNK = PER_W // CHUNK
NBUF = 2

assert PER_W * NW == N
assert NCHUNK * CHUNK == PER_W
assert NCHUNK % NBUF == 0

_MESH = plsc.VectorSubcoreMesh(core_axis_name="c", subcore_axis_name="s")


@functools.partial(
    pl.kernel,
    mesh=_MESH,
    out_type=jax.ShapeDtypeStruct((N, DIM), jnp.float32),
    compiler_params=pltpu.CompilerParams(use_tc_tiling_on_sc=False),
    scratch_types=[
        pltpu.VMEM((NBUF, GPC, G), jnp.int32),
        pltpu.VMEM((NBUF, CHUNK, DIM), jnp.float32),
        pltpu.SemaphoreType.DMA,
        pltpu.SemaphoreType.DMA,
        pltpu.SemaphoreType.DMA,
    ],
)
def _sc_gather(idx_hbm, table_hbm, out_hbm, idx_v, rows_v, gsem, osem0, osem1):
    wid = lax.axis_index("s") * NC + lax.axis_index("c")
    base = wid * PER_W
    osems = (osem0, osem1)

    def store_copy(c, slot):
        return pltpu.make_async_copy(
            rows_v.at[slot],
            out_hbm.at[pl.ds(base + c * CHUNK, CHUNK)],
            osems[slot],
        )

    def do_chunk(c, slot, first):
        if not first:
            store_copy(c - NBUF, slot).wait()
        for j in range(GPC):
            pltpu.sync_copy(
                idx_hbm.at[pl.ds(base + c * CHUNK + j * G, G)],
                idx_v.at[slot, j],
            )
        handles = [
            pltpu.async_copy(
                table_hbm.at[idx_v.at[slot, j]],
                rows_v.at[slot, pl.ds(j * G, G)],
                gsem,
            )
            for j in range(GPC)
        ]
        for h in handles:
            h.wait()
        store_copy(c, slot).start()

    for b in range(NBUF):
        do_chunk(b, b, first=True)

    def body(i, carry):
        c0 = i * NBUF
        for b in range(NBUF):
            do_chunk(c0 + b, b, first=False)
        return carry

    lax.fori_loop(1, NCHUNK // NBUF, body, 0)

    for b in range(NBUF):
        store_copy(NCHUNK - NBUF + b, b).wait()


def kernel(x, table):
    idx = x.reshape(N).astype(jnp.int32)
    out = _sc_gather(idx, table)
    return out.reshape(BATCH, HIST, DIM)


# speedup vs baseline: 1.0567x; 1.0567x over previous
import functools

import jax
import jax.numpy as jnp
from jax import lax
from jax.experimental import pallas as pl
from jax.experimental.pallas import tpu as pltpu
from jax.experimental.pallas import tpu_sc as plsc

BATCH = 4096
HIST = 200
DIM = 64
N = BATCH * HIST

NC = 2
NS = 16
NW = NC * NS
PER_W = N // NW

CHUNK = 800
NCHUNK = PER_W // CHUNK

assert PER_W * NW == N
assert NCHUNK * CHUNK == PER_W
assert NCHUNK % 2 == 0

_MESH = plsc.VectorSubcoreMesh(core_axis_name="c", subcore_axis_name="s")


@functools.partial(
    pl.kernel,
    mesh=_MESH,
    out_type=jax.ShapeDtypeStruct((N, DIM), jnp.float32),
    compiler_params=pltpu.CompilerParams(use_tc_tiling_on_sc=False),
    scratch_types=[
        pltpu.VMEM((NCHUNK, CHUNK), jnp.int32),
        pltpu.VMEM((2, CHUNK, DIM), jnp.float32),
        pltpu.SemaphoreType.DMA,
        pltpu.SemaphoreType.DMA,
        pltpu.SemaphoreType.DMA,
        pltpu.SemaphoreType.DMA,
    ],
)
def _sc_gather(idx_hbm, table_hbm, out_hbm, idx_v, rows_v, g0, g1, o0, o1):
    wid = lax.axis_index("s") * NC + lax.axis_index("c")
    base = wid * PER_W
    gsems = (g0, g1)
    osems = (o0, o1)

    pltpu.sync_copy(idx_hbm.at[pl.ds(wid * NCHUNK, NCHUNK)], idx_v)

    def fire(c, s):
        pltpu.async_copy(table_hbm.at[idx_v.at[c]], rows_v.at[s], gsems[s])

    def drain_gathers(s):
        pltpu.make_async_copy(
            table_hbm.at[pl.ds(0, CHUNK)], rows_v.at[s], gsems[s]
        ).wait()

    def store(c, s):
        return pltpu.make_async_copy(
            rows_v.at[s],
            out_hbm.at[pl.ds(base + c * CHUNK, CHUNK)],
            osems[s],
        )

    fire(0, 0)
    fire(1, 1)
    drain_gathers(0)
    store(0, 0).start()

    def body(i, carry):
        c0 = 2 * i + 2
        for b in range(2):
            c = c0 + b
            store(c - 2, b).wait()
            fire(c, b)
            drain_gathers(1 - b)
            store(c - 1, 1 - b).start()
        return carry

    lax.fori_loop(0, (NCHUNK - 2) // 2, body, 0)

    drain_gathers((NCHUNK - 1) % 2)
    store(NCHUNK - 1, (NCHUNK - 1) % 2).start()
    store(NCHUNK - 2, (NCHUNK - 2) % 2).wait()
    store(NCHUNK - 1, (NCHUNK - 1) % 2).wait()


def kernel(x, table):
    idx = x.reshape(N // CHUNK, CHUNK).astype(jnp.int32)
    out = _sc_gather(idx, table)
    return out.reshape(BATCH, HIST, DIM)
